# Initial kernel scaffold; baseline (speedup 1.0000x reference)
import jax
import jax.numpy as jnp
from jax.experimental import pallas as pl


def kernel(p0, x0, o0, params):
    raise NotImplementedError("write your pallas kernel here")


# speedup vs baseline: 2.9212x; 2.9212x over previous
import functools

import numpy as np
import jax
import jax.numpy as jnp
from jax import lax
from jax.experimental import pallas as pl
from jax.experimental.pallas import tpu as pltpu
from jax.experimental.pallas import tpu_sc as plsc

_PLANES = [32, 64, 128, 256, 512]
_NSAMPLE = [8, 16, 16, 16, 16]
_STRIDE = [1, 4, 4, 4, 4]
_SHARE = 8
_EPS = 1e-5
_NW = 32


def _sc_gather(table, idx):
    B = idx.shape[0]
    D = table.shape[1]
    bpw = B // _NW
    nchunks = bpw // 128
    idx2d = idx.reshape(B // 128, 128)
    mesh = plsc.VectorSubcoreMesh(core_axis_name="c", subcore_axis_name="s")

    @functools.partial(
        pl.kernel,
        mesh=mesh,
        compiler_params=pltpu.CompilerParams(use_tc_tiling_on_sc=False),
        out_type=jax.ShapeDtypeStruct((B, D), jnp.float32),
        scratch_types=[
            pltpu.VMEM((nchunks, 128), jnp.int32),
            pltpu.VMEM((128, D), jnp.float32),
            pltpu.SemaphoreType.DMA,
        ],
    )
    def k(table_hbm, idx_hbm, out_hbm, idx_v, rows_v, sem):
        wid = lax.axis_index("s") * 2 + lax.axis_index("c")
        row0 = wid * nchunks
        pltpu.sync_copy(idx_hbm.at[pl.ds(row0, nchunks), :], idx_v)
        for j in range(nchunks):
            pltpu.async_copy(table_hbm.at[idx_v.at[j]], rows_v, sem).wait()
            pltpu.sync_copy(rows_v, out_hbm.at[pl.ds((row0 + j) * 128, 128), :])

    return k(table, idx2d)


def _gather_rows(table, idx_flat):
    B0 = idx_flat.shape[0]
    B = ((B0 + 4095) // 4096) * 4096
    idxp = idx_flat.astype(jnp.int32)
    if B != B0:
        idxp = jnp.pad(idxp, (0, B - B0))
    out = _sc_gather(table, idxp)
    if B != B0:
        out = out[:B0]
    return out


def _knn(qp, rp, K):
    nq = qp.shape[0]
    nr = rp.shape[0]
    qb = min(256, nq)
    rT = rp.T

    def body(q_ref, rT_ref, idx_ref, dist_ref):
        q = q_ref[...]
        rt = rT_ref[...]
        qr = jnp.dot(q.astype(jnp.bfloat16), rt.astype(jnp.bfloat16),
                     preferred_element_type=jnp.float32)
        sq_q = jnp.sum(q * q, axis=1, keepdims=True)
        sq_r = jnp.sum(rt * rt, axis=0, keepdims=True)
        d = sq_q - 2.0 * qr + sq_r
        iota = lax.broadcasted_iota(jnp.int32, (qb, nr), 1)
        idx_cols = []
        dist_cols = []
        for _ in range(K):
            m = jnp.min(d, axis=1, keepdims=True)
            am = jnp.min(jnp.where(d == m, iota, nr), axis=1, keepdims=True)
            idx_cols.append(am)
            dist_cols.append(m)
            d = jnp.where(iota == am, jnp.inf, d)
        idx_ref[...] = jnp.concatenate(idx_cols, axis=1)
        dist_ref[...] = jnp.concatenate(dist_cols, axis=1)

    idx, dist = pl.pallas_call(
        body,
        grid=(nq // qb,),
        in_specs=[
            pl.BlockSpec((qb, 16), lambda i: (i, 0)),
            pl.BlockSpec((16, nr), lambda i: (0, 0)),
        ],
        out_specs=[
            pl.BlockSpec((qb, K), lambda i: (i, 0)),
            pl.BlockSpec((qb, K), lambda i: (i, 0)),
        ],
        out_shape=[
            jax.ShapeDtypeStruct((nq, K), jnp.int32),
            jax.ShapeDtypeStruct((nq, K), jnp.float32),
        ],
    )(qp, rT)
    return idx, dist


def _mm(a, b):
    return jnp.dot(a.astype(jnp.bfloat16), b.astype(jnp.bfloat16),
                   preferred_element_type=jnp.float32)


def _mmx(a, b):
    return jnp.dot(a, b, precision=lax.Precision.HIGHEST)

def _bn_full(y):
    m = jnp.mean(y, axis=0, keepdims=True)
    v = jnp.mean((y - m) * (y - m), axis=0, keepdims=True)
    return (y - m) / jnp.sqrt(v + _EPS)


def _dbr(x, w, b):
    n = x.shape[0]
    co = w.shape[1]
    bb = jnp.zeros((1, co), jnp.float32) if b is None else b.reshape(1, co)

    def body(x_ref, w_ref, b_ref, o_ref):
        y = _mm(x_ref[...], w_ref[...]) + b_ref[...]
        o_ref[...] = jnp.maximum(_bn_full(y), 0.0)

    return pl.pallas_call(
        body, out_shape=jax.ShapeDtypeStruct((n, co), jnp.float32)
    )(x, w, bb)


def _qkv(x, l1w, wq, bq, wk, bk, wv, bv):
    n = x.shape[0]
    c = l1w.shape[1]

    def body(x_ref, l1_ref, wq_ref, bq_ref, wk_ref, bk_ref, wv_ref, bv_ref,
             q_ref, k_ref, v_ref):
        y = jnp.maximum(_bn_full(_mm(x_ref[...], l1_ref[...])), 0.0)
        q_ref[...] = _mm(y, wq_ref[...]) + bq_ref[...]
        k_ref[...] = _mm(y, wk_ref[...]) + bk_ref[...]
        v_ref[...] = _mm(y, wv_ref[...]) + bv_ref[...]

    return pl.pallas_call(
        body,
        out_shape=[jax.ShapeDtypeStruct((n, c), jnp.float32)] * 3,
    )(x, l1w, wq, bq.reshape(1, c), wk, bk.reshape(1, c), wv, bv.reshape(1, c))


def _stats_init_accum(s_ref, part):
    @pl.when(pl.program_id(0) == 0)
    def _():
        s_ref[...] = jnp.zeros_like(s_ref)
    s_ref[...] += part


def _mv_from_stats(s, cnt):
    del cnt
    m = s[0:1, :]
    return m, 1.0 / jnp.sqrt(s[1:2, :] + _EPS)


def _mv_from_sums(s, cnt):
    m = s[0:1, :] / cnt
    v = s[1:2, :] / cnt - m * m
    return m, 1.0 / jnp.sqrt(v + _EPS)


def _group_mat(k, c):
    g = np.zeros((k * c, c), np.float32)
    for j in range(k):
        for ch in range(c):
            g[j * c + ch, ch] = 1.0
    return g


def _bn_stats3(x3):
    n, k, c = x3.shape
    cnt = float(n * k)
    if c >= 128:
        def body(x_ref, s_ref):
            x2 = x_ref[...].reshape(n * k, c)
            m = jnp.mean(x2, axis=0, keepdims=True)
            v = jnp.mean((x2 - m) * (x2 - m), axis=0, keepdims=True)
            s_ref[...] = jnp.concatenate([m, v], 0)

        return pl.pallas_call(
            body, out_shape=jax.ShapeDtypeStruct((2, c), jnp.float32)
        )(x3)
    g_np = _group_mat(k, c)
    g, gt = jnp.asarray(g_np), jnp.asarray(g_np.T.copy())
    xf = x3.reshape(n, k * c)

    def body(x_ref, g_ref, gt_ref, s_ref):
        xx = x_ref[...]
        m = _mmx(jnp.sum(xx, axis=0, keepdims=True), g_ref[...]) / cnt
        d = xx - _mmx(m, gt_ref[...])
        v = _mmx(jnp.sum(d * d, axis=0, keepdims=True), g_ref[...]) / cnt
        s_ref[...] = jnp.concatenate([m, v], 0)

    return pl.pallas_call(
        body, out_shape=jax.ShapeDtypeStruct((2, c), jnp.float32)
    )(xf, g, gt)


def _pt_layer(p16, pg3, xq, xkg, xvg, prm, n, k, c, nb):
    blk = n // nb
    cnt = float(n * k)
    c8 = c // _SHARE
    p1w = jnp.zeros((16, 16), jnp.float32).at[:3, :3].set(prm["p1"]["w"])
    b1 = jnp.zeros((1, 16), jnp.float32).at[0, :3].set(prm["p1"]["b"])
    p2w = jnp.zeros((16, c), jnp.float32).at[:3, :].set(prm["p2"]["w"])
    b2 = prm["p2"]["b"].reshape(1, c)
    w1w, w1b = prm["w1"]["w"], prm["w1"]["b"].reshape(1, c8)
    w2w, w2b = prm["w2"]["w"], prm["w2"]["b"].reshape(1, c8)
    e_np = np.zeros((c8, c), np.float32)
    for r in range(c8):
        for s in range(_SHARE):
            e_np[r, s * c8 + r] = 1.0
    emat = jnp.asarray(e_np)

    spec_g3 = lambda d: pl.BlockSpec((blk, k, d), lambda i: (i, 0, 0))
    spec_2d = lambda d: pl.BlockSpec((blk, d), lambda i: (i, 0))
    spec_w = lambda a, b: pl.BlockSpec((a, b), lambda i: (0, 0))
    spec_s = lambda d: pl.BlockSpec((2, d), lambda i: (0, 0))

    p1bd = jnp.kron(jnp.eye(k, dtype=jnp.float32), p1w)
    b1t = jnp.tile(b1, (1, k))
    g16_np = _group_mat(k, 16)
    g16, g16t = jnp.asarray(g16_np), jnp.asarray(g16_np.T.copy())
    pgf = pg3.reshape(n, k * 16)
    ptil = jnp.tile(p16, (1, k))

    def s1_body(pg_ref, p_ref, w_ref, b_ref, g_ref, gt_ref, s_ref):
        t = _mm(pg_ref[...] - p_ref[...], w_ref[...]) + b_ref[...]
        m = _mmx(jnp.sum(t, axis=0, keepdims=True), g_ref[...]) / cnt
        d = t - _mmx(m, gt_ref[...])
        v = _mmx(jnp.sum(d * d, axis=0, keepdims=True), g_ref[...]) / cnt
        s_ref[...] = jnp.concatenate([m, v], 0)

    s1 = pl.pallas_call(
        s1_body,
        out_shape=jax.ShapeDtypeStruct((2, 16), jnp.float32),
    )(pgf, ptil, p1bd, b1t, g16, g16t)

    def s2_body(pg_ref, p_ref, xkg_ref, xq_ref, p1_ref, b1_ref, s1_ref,
                p2_ref, b2_ref, pe_ref, w_ref):
        pr = (pg_ref[...] - p_ref[...][:, None, :]).reshape(blk * k, 16)
        t = _mm(pr, p1_ref[...]) + b1_ref[...]
        m, rs = _mv_from_stats(s1_ref[...], cnt)
        t = jnp.maximum((t - m) * rs, 0.0)
        pe2 = _mm(t, p2_ref[...]) + b2_ref[...]
        pe_ref[...] = pe2.reshape(blk, k, c)
        w_ref[...] = xkg_ref[...] - xq_ref[...][:, None, :] + pe_ref[...]

    pe, wn = pl.pallas_call(
        s2_body, grid=(nb,),
        in_specs=[spec_g3(16), spec_2d(16), spec_g3(c), spec_2d(c),
                  spec_w(16, 16), spec_w(1, 16), spec_s(16),
                  spec_w(16, c), spec_w(1, c)],
        out_specs=[spec_g3(c), spec_g3(c)],
        out_shape=[jax.ShapeDtypeStruct((n, k, c), jnp.float32),
                   jax.ShapeDtypeStruct((n, k, c), jnp.float32)],
    )(pg3, p16, xkg, xq, p1w, b1, s1, p2w, b2)
    s2 = _bn_stats3(wn)

    def s3_body(w_ref, s2_ref, w1_ref, b1_ref, o_ref):
        w2 = w_ref[...].reshape(blk * k, c)
        m, rs = _mv_from_stats(s2_ref[...], cnt)
        w2 = jnp.maximum((w2 - m) * rs, 0.0)
        o = _mm(w2, w1_ref[...]) + b1_ref[...]
        o_ref[...] = o.reshape(blk, k, c8)

    w1out = pl.pallas_call(
        s3_body, grid=(nb,),
        in_specs=[spec_g3(c), spec_s(c), spec_w(c, c8), spec_w(1, c8)],
        out_specs=spec_g3(c8),
        out_shape=jax.ShapeDtypeStruct((n, k, c8), jnp.float32),
    )(wn, s2, w1w, w1b)
    s3 = _bn_stats3(w1out)

    def s4_body(w1o_ref, s3_ref, w2_ref, b2w_ref, xvg_ref, pe_ref, e_ref, o_ref):
        z = w1o_ref[...].reshape(blk * k, c8)
        m, rs = _mv_from_stats(s3_ref[...], cnt)
        z = jnp.maximum((z - m) * rs, 0.0)
        z = _mm(z, w2_ref[...]) + b2w_ref[...]
        z3 = z.reshape(blk, k, c8)
        z3 = z3 - jnp.max(z3, axis=1, keepdims=True)
        ez = jnp.exp(z3)
        att = ez / jnp.sum(ez, axis=1, keepdims=True)
        wfull = _mmx(att.reshape(blk * k, c8), e_ref[...]).reshape(blk, k, c)
        v3 = xvg_ref[...] + pe_ref[...]
        o_ref[...] = jnp.sum(v3 * wfull, axis=1)

    return pl.pallas_call(
        s4_body, grid=(nb,),
        in_specs=[spec_g3(c8), spec_s(c8), spec_w(c8, c8), spec_w(1, c8),
                  spec_g3(c), spec_g3(c), spec_w(c8, c)],
        out_specs=spec_2d(c),
        out_shape=jax.ShapeDtypeStruct((n, c), jnp.float32),
    )(w1out, s3, w2w, w2b, xvg, pe, emat)


def _pt_post(pt_out, idn, l3w):
    n, c = pt_out.shape

    def body(o_ref, x_ref, w_ref, r_ref):
        y = jnp.maximum(_bn_full(o_ref[...]), 0.0)
        z = _bn_full(_mm(y, w_ref[...]))
        r_ref[...] = jnp.maximum(z + x_ref[...], 0.0)

    return pl.pallas_call(
        body, out_shape=jax.ShapeDtypeStruct((n, c), jnp.float32)
    )(pt_out, idn, l3w)


def _pt_block(prm, lvl_ctx, x, nb):
    p16, pg3, idx, n, k = lvl_ctx
    c = x.shape[1]
    xq, xk_all, xv_all = _qkv(
        x, prm["lin1"]["w"], prm["wq"]["w"], prm["wq"]["b"],
        prm["wk"]["w"], prm["wk"]["b"], prm["wv"]["w"], prm["wv"]["b"])
    xkg = _gather_rows(xk_all, idx.reshape(-1)).reshape(n, k, c)
    xvg = _gather_rows(xv_all, idx.reshape(-1)).reshape(n, k, c)
    out = _pt_layer(p16, pg3, xq, xkg, xvg, prm, n, k, c, nb)
    return _pt_post(out, x, prm["lin3"]["w"])


def _tdown(prm, newp16, pgg3, xg3, m, k, ci_pad, co, nb):
    blk = m // nb
    cnt = float(m * k)
    w_full = prm["w"]
    ci = w_full.shape[0] - 3
    wp = jnp.zeros((16, co), jnp.float32).at[:3, :].set(w_full[:3])
    wx = jnp.zeros((ci_pad, co), jnp.float32).at[:ci, :].set(w_full[3:])
    b = prm["b"].reshape(1, co)

    spec_g3 = lambda d: pl.BlockSpec((blk, k, d), lambda i: (i, 0, 0))
    spec_2d = lambda d: pl.BlockSpec((blk, d), lambda i: (i, 0))
    spec_w = lambda a, bb: pl.BlockSpec((a, bb), lambda i: (0, 0))
    spec_s = lambda d: pl.BlockSpec((2, d), lambda i: (0, 0))

    def t_of(pg_ref, p_ref, xg_ref, wp_ref, wx_ref, b_ref):
        pr = (pg_ref[...] - p_ref[...][:, None, :]).reshape(blk * k, 16)
        xg = xg_ref[...].reshape(blk * k, ci_pad)
        return _mm(pr, wp_ref[...]) + _mm(xg, wx_ref[...]) + b_ref[...]

    def s_body(pg_ref, p_ref, xg_ref, wp_ref, wx_ref, b_ref, s_ref):
        t = t_of(pg_ref, p_ref, xg_ref, wp_ref, wx_ref, b_ref)
        part = jnp.concatenate(
            [jnp.sum(t, 0, keepdims=True), jnp.sum(t * t, 0, keepdims=True)], 0)
        _stats_init_accum(s_ref, part)

    st = pl.pallas_call(
        s_body, grid=(nb,),
        in_specs=[spec_g3(16), spec_2d(16), spec_g3(ci_pad),
                  spec_w(16, co), spec_w(ci_pad, co), spec_w(1, co)],
        out_specs=spec_s(co),
        out_shape=jax.ShapeDtypeStruct((2, co), jnp.float32),
    )(pgg3, newp16, xg3, wp, wx, b)

    def a_body(pg_ref, p_ref, xg_ref, wp_ref, wx_ref, b_ref, s_ref, o_ref):
        t = t_of(pg_ref, p_ref, xg_ref, wp_ref, wx_ref, b_ref)
        mm, rs = _mv_from_sums(s_ref[...], cnt)
        t = jnp.maximum((t - mm) * rs, 0.0)
        o_ref[...] = jnp.max(t.reshape(blk, k, co), axis=1)

    return pl.pallas_call(
        a_body, grid=(nb,),
        in_specs=[spec_g3(16), spec_2d(16), spec_g3(ci_pad),
                  spec_w(16, co), spec_w(ci_pad, co), spec_w(1, co), spec_s(co)],
        out_specs=spec_2d(co),
        out_shape=jax.ShapeDtypeStruct((m, co), jnp.float32),
    )(pgg3, newp16, xg3, wp, wx, b, st)


def _tup_interp(y1, y2f, dist3, c):
    n = y1.shape[0]
    e_np = np.zeros((3, 3 * c), np.float32)
    g_np = np.zeros((3 * c, c), np.float32)
    for j in range(3):
        for ch in range(c):
            e_np[j, j * c + ch] = 1.0
            g_np[j * c + ch, ch] = 1.0
    emat, gmat = jnp.asarray(e_np), jnp.asarray(g_np)

    def body(y1_ref, y2_ref, d_ref, e_ref, g_ref, o_ref):
        d = jnp.sqrt(jnp.maximum(d_ref[...], 1e-10))
        w = 1.0 / (d + 1e-8)
        w = w / jnp.sum(w, axis=1, keepdims=True)
        wfull = _mmx(w, e_ref[...])
        o_ref[...] = y1_ref[...] + _mmx(y2_ref[...] * wfull, g_ref[...])

    return pl.pallas_call(
        body, out_shape=jax.ShapeDtypeStruct((n, c), jnp.float32)
    )(y1, y2f, dist3, emat, gmat)


def _tup_head(prm, x):
    n, c = x.shape
    l1w, l1b = prm["l1"]["w"], prm["l1"]["b"].reshape(1, c)
    l2w, l2b = prm["l2"]["w"], prm["l2"]["b"].reshape(1, c)
    wx, wg = l1w[:c], l1w[c:]

    def body(x_ref, wx_ref, wg_ref, b1_ref, w2_ref, b2_ref, o_ref):
        xx = x_ref[...]
        g = jnp.mean(xx, axis=0, keepdims=True)
        g = jnp.maximum(_mm(g, w2_ref[...]) + b2_ref[...], 0.0)
        t = _mm(xx, wx_ref[...]) + _mm(g, wg_ref[...]) + b1_ref[...]
        o_ref[...] = jnp.maximum(_bn_full(t), 0.0)

    return pl.pallas_call(
        body, out_shape=jax.ShapeDtypeStruct((n, c), jnp.float32)
    )(x, wx, wg, l1b, l2w, l2b)


def _cls_head(prm, x):
    n, c = x.shape
    ko = prm["l2"]["w"].shape[1]

    def body(x_ref, w1_ref, b1_ref, w2_ref, b2_ref, o_ref):
        y = jnp.maximum(_bn_full(_mm(x_ref[...], w1_ref[...]) + b1_ref[...]), 0.0)
        o_ref[...] = _mm(y, w2_ref[...]) + b2_ref[...]

    return pl.pallas_call(
        body, out_shape=jax.ShapeDtypeStruct((n, ko), jnp.float32)
    )(x, prm["l1"]["w"], prm["l1"]["b"].reshape(1, c),
      prm["l2"]["w"], prm["l2"]["b"].reshape(1, ko))


_NB_BLOCK = [16, 8, 2, 1, 1]
_NB_TDOWN = [0, 8, 2, 1, 1]


def kernel(p0, x0, o0, params):
    del o0
    P = params
    n0 = p0.shape[0]
    ns = [n0 // (4 ** max(i, 0)) if i == 0 else n0 // (4 ** i) for i in range(5)]

    p16 = jnp.pad(p0, ((0, 0), (0, 13)))
    x = jnp.concatenate([p0, x0], axis=1)

    ps16, xs = [], []
    lvl_ctx = []
    for i in range(5):
        n_i, c_i, k_i = ns[i], _PLANES[i], _NSAMPLE[i]
        if _STRIDE[i] == 1:
            y = _dbr(x, P["enc"][i]["td"]["w"], None)
        else:
            newp16 = p16[::4]
            gi, _ = _knn(newp16, p16, k_i)
            pgg = _gather_rows(p16, gi.reshape(-1)).reshape(n_i, k_i, 16)
            ci = x.shape[1]
            ci_pad = ((ci + 15) // 16) * 16
            x_pad = jnp.pad(x, ((0, 0), (0, ci_pad - ci))) if ci_pad != ci else x
            xg = _gather_rows(x_pad, gi.reshape(-1)).reshape(n_i, k_i, ci_pad)
            y = _tdown(P["enc"][i]["td"], newp16, pgg, xg, n_i, k_i,
                       ci_pad, c_i, _NB_TDOWN[i])
            p16 = newp16
        idx, _ = _knn(p16, p16, k_i)
        pg3 = _gather_rows(p16, idx.reshape(-1)).reshape(n_i, k_i, 16)
        ctx = (p16, pg3, idx, n_i, k_i)
        lvl_ctx.append(ctx)
        for b in P["enc"][i]["blocks"]:
            y = _pt_block(b, ctx, y, _NB_BLOCK[i])
        ps16.append(p16)
        xs.append(y)
        x = y

    x = _tup_head(P["dec"][4]["tu"], xs[4])
    for b in P["dec"][4]["blocks"]:
        x = _pt_block(b, lvl_ctx[4], x, _NB_BLOCK[4])
    xs[4] = x

    for i in (3, 2, 1, 0):
        c_i = _PLANES[i]
        tu = P["dec"][i]["tu"]
        y1 = _dbr(xs[i], tu["l1"]["w"], tu["l1"]["b"])
        y2 = _dbr(xs[i + 1], tu["l2"]["w"], tu["l2"]["b"])
        ii, dd = _knn(ps16[i], ps16[i + 1], 3)
        y2f = _gather_rows(y2, ii.reshape(-1)).reshape(ns[i], 3 * c_i)
        x = _tup_interp(y1, y2f, dd, c_i)
        for b in P["dec"][i]["blocks"]:
            x = _pt_block(b, lvl_ctx[i], x, _NB_BLOCK[i])
        xs[i] = x

    return _cls_head(P["cls"], xs[0])


# speedup vs baseline: 3.0996x; 1.0610x over previous
import functools

import numpy as np
import jax
import jax.numpy as jnp
from jax import lax
from jax.experimental import pallas as pl
from jax.experimental.pallas import tpu as pltpu
from jax.experimental.pallas import tpu_sc as plsc

_PLANES = [32, 64, 128, 256, 512]
_NSAMPLE = [8, 16, 16, 16, 16]
_STRIDE = [1, 4, 4, 4, 4]
_SHARE = 8
_EPS = 1e-5
_NW = 32


def _sc_gather(table, idx):
    B = idx.shape[0]
    D = table.shape[1]
    bpw = B // _NW
    nchunks = bpw // 128
    idx2d = idx.reshape(B // 128, 128)
    mesh = plsc.VectorSubcoreMesh(core_axis_name="c", subcore_axis_name="s")

    @functools.partial(
        pl.kernel,
        mesh=mesh,
        compiler_params=pltpu.CompilerParams(use_tc_tiling_on_sc=False),
        out_type=jax.ShapeDtypeStruct((B, D), jnp.float32),
        scratch_types=[
            pltpu.VMEM((nchunks, 128), jnp.int32),
            pltpu.VMEM((128, D), jnp.float32),
            pltpu.SemaphoreType.DMA,
        ],
    )
    def k(table_hbm, idx_hbm, out_hbm, idx_v, rows_v, sem):
        wid = lax.axis_index("s") * 2 + lax.axis_index("c")
        row0 = wid * nchunks
        pltpu.sync_copy(idx_hbm.at[pl.ds(row0, nchunks), :], idx_v)
        for j in range(nchunks):
            pltpu.async_copy(table_hbm.at[idx_v.at[j]], rows_v, sem).wait()
            pltpu.sync_copy(rows_v, out_hbm.at[pl.ds((row0 + j) * 128, 128), :])

    return k(table, idx2d)


def _gather_rows(table, idx_flat):
    B0 = idx_flat.shape[0]
    B = ((B0 + 4095) // 4096) * 4096
    idxp = idx_flat.astype(jnp.int32)
    if B != B0:
        idxp = jnp.pad(idxp, (0, B - B0))
    out = _sc_gather(table, idxp)
    if B != B0:
        out = out[:B0]
    return out


def _knn(qp, rp, K):
    nq = qp.shape[0]
    nr = rp.shape[0]
    qb = min(256, nq)
    rT = rp.T

    def body(q_ref, rT_ref, idx_ref, dist_ref):
        q = q_ref[...]
        rt = rT_ref[...]
        qr = jnp.dot(q.astype(jnp.bfloat16), rt.astype(jnp.bfloat16),
                     preferred_element_type=jnp.float32)
        sq_q = jnp.sum(q * q, axis=1, keepdims=True)
        sq_r = jnp.sum(rt * rt, axis=0, keepdims=True)
        d = sq_q - 2.0 * qr + sq_r
        iota = lax.broadcasted_iota(jnp.int32, (qb, nr), 1)
        idx_cols = []
        dist_cols = []
        for _ in range(K):
            m = jnp.min(d, axis=1, keepdims=True)
            am = jnp.min(jnp.where(d == m, iota, nr), axis=1, keepdims=True)
            idx_cols.append(am)
            dist_cols.append(m)
            d = jnp.where(iota == am, jnp.inf, d)
        idx_ref[...] = jnp.concatenate(idx_cols, axis=1)
        dist_ref[...] = jnp.concatenate(dist_cols, axis=1)

    idx, dist = pl.pallas_call(
        body,
        grid=(nq // qb,),
        in_specs=[
            pl.BlockSpec((qb, 16), lambda i: (i, 0)),
            pl.BlockSpec((16, nr), lambda i: (0, 0)),
        ],
        out_specs=[
            pl.BlockSpec((qb, K), lambda i: (i, 0)),
            pl.BlockSpec((qb, K), lambda i: (i, 0)),
        ],
        out_shape=[
            jax.ShapeDtypeStruct((nq, K), jnp.int32),
            jax.ShapeDtypeStruct((nq, K), jnp.float32),
        ],
    )(qp, rT)
    return idx, dist


def _mm(a, b):
    return jnp.dot(a.astype(jnp.bfloat16), b.astype(jnp.bfloat16),
                   preferred_element_type=jnp.float32)


def _mmx(a, b):
    return jnp.dot(a, b, precision=lax.Precision.HIGHEST)

def _bn_full(y):
    m = jnp.mean(y, axis=0, keepdims=True)
    v = jnp.mean((y - m) * (y - m), axis=0, keepdims=True)
    return (y - m) * lax.rsqrt(v + _EPS)


def _dbr(x, w, b):
    n = x.shape[0]
    co = w.shape[1]
    bb = jnp.zeros((1, co), jnp.float32) if b is None else b.reshape(1, co)

    def body(x_ref, w_ref, b_ref, o_ref):
        y = _mm(x_ref[...], w_ref[...]) + b_ref[...]
        o_ref[...] = jnp.maximum(_bn_full(y), 0.0)

    return pl.pallas_call(
        body, out_shape=jax.ShapeDtypeStruct((n, co), jnp.float32)
    )(x, w, bb)


def _qkv(x, l1w, wq, bq, wk, bk, wv, bv):
    n = x.shape[0]
    c = l1w.shape[1]

    def body(x_ref, l1_ref, wq_ref, bq_ref, wk_ref, bk_ref, wv_ref, bv_ref,
             q_ref, k_ref, v_ref):
        y = jnp.maximum(_bn_full(_mm(x_ref[...], l1_ref[...])), 0.0)
        q_ref[...] = _mm(y, wq_ref[...]) + bq_ref[...]
        k_ref[...] = _mm(y, wk_ref[...]) + bk_ref[...]
        v_ref[...] = _mm(y, wv_ref[...]) + bv_ref[...]

    return pl.pallas_call(
        body,
        out_shape=[jax.ShapeDtypeStruct((n, c), jnp.float32)] * 3,
    )(x, l1w, wq, bq.reshape(1, c), wk, bk.reshape(1, c), wv, bv.reshape(1, c))


def _stats_init_accum(s_ref, part):
    @pl.when(pl.program_id(0) == 0)
    def _():
        s_ref[...] = jnp.zeros_like(s_ref)
    s_ref[...] += part


def _mv_from_stats(s, cnt):
    m = s[0:1, :] / cnt
    v = s[1:2, :] / cnt - m * m
    return m, lax.rsqrt(v + _EPS)


def _pt_layer(p16, pg3, xq, xkg, xvg, prm, n, k, c, nb):
    blk = n // nb
    cnt = float(n * k)
    c8 = c // _SHARE
    p1w = jnp.zeros((16, 16), jnp.float32).at[:3, :3].set(prm["p1"]["w"])
    b1 = jnp.zeros((1, 16), jnp.float32).at[0, :3].set(prm["p1"]["b"])
    p2w = jnp.zeros((16, c), jnp.float32).at[:3, :].set(prm["p2"]["w"])
    b2 = prm["p2"]["b"].reshape(1, c)
    w1w, w1b = prm["w1"]["w"], prm["w1"]["b"].reshape(1, c8)
    w2w, w2b = prm["w2"]["w"], prm["w2"]["b"].reshape(1, c8)
    e_np = np.zeros((c8, c), np.float32)
    for r in range(c8):
        for s in range(_SHARE):
            e_np[r, s * c8 + r] = 1.0
    emat = jnp.asarray(e_np)

    spec_g3 = lambda d: pl.BlockSpec((blk, k, d), lambda i: (i, 0, 0))
    spec_2d = lambda d: pl.BlockSpec((blk, d), lambda i: (i, 0))
    spec_w = lambda a, b: pl.BlockSpec((a, b), lambda i: (0, 0))
    spec_s = lambda d: pl.BlockSpec((2, d), lambda i: (0, 0))

    def s1_body(pg_ref, p_ref, w_ref, b_ref, s_ref):
        pr = (pg_ref[...] - p_ref[...][:, None, :]).reshape(blk * k, 16)
        t = _mm(pr, w_ref[...]) + b_ref[...]
        part = jnp.concatenate(
            [jnp.sum(t, 0, keepdims=True), jnp.sum(t * t, 0, keepdims=True)], 0)
        _stats_init_accum(s_ref, part)

    s1 = pl.pallas_call(
        s1_body, grid=(nb,),
        in_specs=[spec_g3(16), spec_2d(16), spec_w(16, 16), spec_w(1, 16)],
        out_specs=spec_s(16),
        out_shape=jax.ShapeDtypeStruct((2, 16), jnp.float32),
    )(pg3, p16, p1w, b1)

    def s2_body(pg_ref, p_ref, xkg_ref, xq_ref, p1_ref, b1_ref, s1_ref,
                p2_ref, b2_ref, pe_ref, s_ref):
        pr = (pg_ref[...] - p_ref[...][:, None, :]).reshape(blk * k, 16)
        t = _mm(pr, p1_ref[...]) + b1_ref[...]
        m, rs = _mv_from_stats(s1_ref[...], cnt)
        t = jnp.maximum((t - m) * rs, 0.0)
        pe2 = _mm(t, p2_ref[...]) + b2_ref[...]
        pe_ref[...] = pe2.reshape(blk, k, c)
        w3 = xkg_ref[...] - xq_ref[...][:, None, :] + pe_ref[...]
        w2 = w3.reshape(blk * k, c)
        part = jnp.concatenate(
            [jnp.sum(w2, 0, keepdims=True), jnp.sum(w2 * w2, 0, keepdims=True)], 0)
        _stats_init_accum(s_ref, part)

    pe, s2 = pl.pallas_call(
        s2_body, grid=(nb,),
        in_specs=[spec_g3(16), spec_2d(16), spec_g3(c), spec_2d(c),
                  spec_w(16, 16), spec_w(1, 16), spec_s(16),
                  spec_w(16, c), spec_w(1, c)],
        out_specs=[spec_g3(c), spec_s(c)],
        out_shape=[jax.ShapeDtypeStruct((n, k, c), jnp.float32),
                   jax.ShapeDtypeStruct((2, c), jnp.float32)],
    )(pg3, p16, xkg, xq, p1w, b1, s1, p2w, b2)

    def s3_body(xkg_ref, xq_ref, pe_ref, s2_ref, w1_ref, b1_ref, o_ref, s_ref):
        w3 = xkg_ref[...] - xq_ref[...][:, None, :] + pe_ref[...]
        w2 = w3.reshape(blk * k, c)
        m, rs = _mv_from_stats(s2_ref[...], cnt)
        w2 = jnp.maximum((w2 - m) * rs, 0.0)
        o = _mm(w2, w1_ref[...]) + b1_ref[...]
        o_ref[...] = o.reshape(blk, k, c8)
        part = jnp.concatenate(
            [jnp.sum(o, 0, keepdims=True), jnp.sum(o * o, 0, keepdims=True)], 0)
        _stats_init_accum(s_ref, part)

    w1out, s3 = pl.pallas_call(
        s3_body, grid=(nb,),
        in_specs=[spec_g3(c), spec_2d(c), spec_g3(c), spec_s(c),
                  spec_w(c, c8), spec_w(1, c8)],
        out_specs=[spec_g3(c8), spec_s(c8)],
        out_shape=[jax.ShapeDtypeStruct((n, k, c8), jnp.float32),
                   jax.ShapeDtypeStruct((2, c8), jnp.float32)],
    )(xkg, xq, pe, s2, w1w, w1b)

    def s4_body(w1o_ref, s3_ref, w2_ref, b2w_ref, xvg_ref, pe_ref, e_ref, o_ref):
        z = w1o_ref[...].reshape(blk * k, c8)
        m, rs = _mv_from_stats(s3_ref[...], cnt)
        z = jnp.maximum((z - m) * rs, 0.0)
        z = _mm(z, w2_ref[...]) + b2w_ref[...]
        z3 = z.reshape(blk, k, c8)
        z3 = z3 - jnp.max(z3, axis=1, keepdims=True)
        ez = jnp.exp(z3)
        att = ez / jnp.sum(ez, axis=1, keepdims=True)
        wfull = _mmx(att.reshape(blk * k, c8), e_ref[...]).reshape(blk, k, c)
        v3 = xvg_ref[...] + pe_ref[...]
        o_ref[...] = jnp.sum(v3 * wfull, axis=1)

    return pl.pallas_call(
        s4_body, grid=(nb,),
        in_specs=[spec_g3(c8), spec_s(c8), spec_w(c8, c8), spec_w(1, c8),
                  spec_g3(c), spec_g3(c), spec_w(c8, c)],
        out_specs=spec_2d(c),
        out_shape=jax.ShapeDtypeStruct((n, c), jnp.float32),
    )(w1out, s3, w2w, w2b, xvg, pe, emat)


def _pt_post(pt_out, idn, l3w):
    n, c = pt_out.shape

    def body(o_ref, x_ref, w_ref, r_ref):
        y = jnp.maximum(_bn_full(o_ref[...]), 0.0)
        z = _bn_full(_mm(y, w_ref[...]))
        r_ref[...] = jnp.maximum(z + x_ref[...], 0.0)

    return pl.pallas_call(
        body, out_shape=jax.ShapeDtypeStruct((n, c), jnp.float32)
    )(pt_out, idn, l3w)


def _pt_block(prm, lvl_ctx, x, nb):
    p16, pg3, idx, n, k = lvl_ctx
    c = x.shape[1]
    xq, xk_all, xv_all = _qkv(
        x, prm["lin1"]["w"], prm["wq"]["w"], prm["wq"]["b"],
        prm["wk"]["w"], prm["wk"]["b"], prm["wv"]["w"], prm["wv"]["b"])
    xkg = _gather_rows(xk_all, idx.reshape(-1)).reshape(n, k, c)
    xvg = _gather_rows(xv_all, idx.reshape(-1)).reshape(n, k, c)
    out = _pt_layer(p16, pg3, xq, xkg, xvg, prm, n, k, c, nb)
    return _pt_post(out, x, prm["lin3"]["w"])


def _tdown(prm, newp16, pgg3, xg3, m, k, ci_pad, co, nb):
    blk = m // nb
    cnt = float(m * k)
    w_full = prm["w"]
    ci = w_full.shape[0] - 3
    wp = jnp.zeros((16, co), jnp.float32).at[:3, :].set(w_full[:3])
    wx = jnp.zeros((ci_pad, co), jnp.float32).at[:ci, :].set(w_full[3:])
    b = prm["b"].reshape(1, co)

    spec_g3 = lambda d: pl.BlockSpec((blk, k, d), lambda i: (i, 0, 0))
    spec_2d = lambda d: pl.BlockSpec((blk, d), lambda i: (i, 0))
    spec_w = lambda a, bb: pl.BlockSpec((a, bb), lambda i: (0, 0))
    spec_s = lambda d: pl.BlockSpec((2, d), lambda i: (0, 0))

    def t_of(pg_ref, p_ref, xg_ref, wp_ref, wx_ref, b_ref):
        pr = (pg_ref[...] - p_ref[...][:, None, :]).reshape(blk * k, 16)
        xg = xg_ref[...].reshape(blk * k, ci_pad)
        return _mm(pr, wp_ref[...]) + _mm(xg, wx_ref[...]) + b_ref[...]

    def s_body(pg_ref, p_ref, xg_ref, wp_ref, wx_ref, b_ref, s_ref):
        t = t_of(pg_ref, p_ref, xg_ref, wp_ref, wx_ref, b_ref)
        part = jnp.concatenate(
            [jnp.sum(t, 0, keepdims=True), jnp.sum(t * t, 0, keepdims=True)], 0)
        _stats_init_accum(s_ref, part)

    st = pl.pallas_call(
        s_body, grid=(nb,),
        in_specs=[spec_g3(16), spec_2d(16), spec_g3(ci_pad),
                  spec_w(16, co), spec_w(ci_pad, co), spec_w(1, co)],
        out_specs=spec_s(co),
        out_shape=jax.ShapeDtypeStruct((2, co), jnp.float32),
    )(pgg3, newp16, xg3, wp, wx, b)

    def a_body(pg_ref, p_ref, xg_ref, wp_ref, wx_ref, b_ref, s_ref, o_ref):
        t = t_of(pg_ref, p_ref, xg_ref, wp_ref, wx_ref, b_ref)
        mm, rs = _mv_from_stats(s_ref[...], cnt)
        t = jnp.maximum((t - mm) * rs, 0.0)
        o_ref[...] = jnp.max(t.reshape(blk, k, co), axis=1)

    return pl.pallas_call(
        a_body, grid=(nb,),
        in_specs=[spec_g3(16), spec_2d(16), spec_g3(ci_pad),
                  spec_w(16, co), spec_w(ci_pad, co), spec_w(1, co), spec_s(co)],
        out_specs=spec_2d(co),
        out_shape=jax.ShapeDtypeStruct((m, co), jnp.float32),
    )(pgg3, newp16, xg3, wp, wx, b, st)


def _tup_interp(y1, y2f, dist3, c):
    n = y1.shape[0]
    e_np = np.zeros((3, 3 * c), np.float32)
    g_np = np.zeros((3 * c, c), np.float32)
    for j in range(3):
        for ch in range(c):
            e_np[j, j * c + ch] = 1.0
            g_np[j * c + ch, ch] = 1.0
    emat, gmat = jnp.asarray(e_np), jnp.asarray(g_np)

    def body(y1_ref, y2_ref, d_ref, e_ref, g_ref, o_ref):
        d = jnp.sqrt(jnp.maximum(d_ref[...], 1e-10))
        w = 1.0 / (d + 1e-8)
        w = w / jnp.sum(w, axis=1, keepdims=True)
        wfull = _mmx(w, e_ref[...])
        o_ref[...] = y1_ref[...] + _mmx(y2_ref[...] * wfull, g_ref[...])

    return pl.pallas_call(
        body, out_shape=jax.ShapeDtypeStruct((n, c), jnp.float32)
    )(y1, y2f, dist3, emat, gmat)


def _tup_head(prm, x):
    n, c = x.shape
    l1w, l1b = prm["l1"]["w"], prm["l1"]["b"].reshape(1, c)
    l2w, l2b = prm["l2"]["w"], prm["l2"]["b"].reshape(1, c)
    wx, wg = l1w[:c], l1w[c:]

    def body(x_ref, wx_ref, wg_ref, b1_ref, w2_ref, b2_ref, o_ref):
        xx = x_ref[...]
        g = jnp.mean(xx, axis=0, keepdims=True)
        g = jnp.maximum(_mm(g, w2_ref[...]) + b2_ref[...], 0.0)
        t = _mm(xx, wx_ref[...]) + _mm(g, wg_ref[...]) + b1_ref[...]
        o_ref[...] = jnp.maximum(_bn_full(t), 0.0)

    return pl.pallas_call(
        body, out_shape=jax.ShapeDtypeStruct((n, c), jnp.float32)
    )(x, wx, wg, l1b, l2w, l2b)


def _cls_head(prm, x):
    n, c = x.shape
    ko = prm["l2"]["w"].shape[1]

    def body(x_ref, w1_ref, b1_ref, w2_ref, b2_ref, o_ref):
        y = jnp.maximum(_bn_full(_mm(x_ref[...], w1_ref[...]) + b1_ref[...]), 0.0)
        o_ref[...] = _mm(y, w2_ref[...]) + b2_ref[...]

    return pl.pallas_call(
        body, out_shape=jax.ShapeDtypeStruct((n, ko), jnp.float32)
    )(x, prm["l1"]["w"], prm["l1"]["b"].reshape(1, c),
      prm["l2"]["w"], prm["l2"]["b"].reshape(1, ko))


_NB_BLOCK = [16, 8, 2, 1, 1]
_NB_TDOWN = [0, 8, 2, 1, 1]


def kernel(p0, x0, o0, params):
    del o0
    P = params
    n0 = p0.shape[0]
    ns = [n0 // (4 ** max(i, 0)) if i == 0 else n0 // (4 ** i) for i in range(5)]

    p16 = jnp.pad(p0, ((0, 0), (0, 13)))
    x = jnp.concatenate([p0, x0], axis=1)

    ps16, xs = [], []
    lvl_ctx = []
    for i in range(5):
        n_i, c_i, k_i = ns[i], _PLANES[i], _NSAMPLE[i]
        if _STRIDE[i] == 1:
            y = _dbr(x, P["enc"][i]["td"]["w"], None)
        else:
            newp16 = p16[::4]
            gi, _ = _knn(newp16, p16, k_i)
            pgg = _gather_rows(p16, gi.reshape(-1)).reshape(n_i, k_i, 16)
            ci = x.shape[1]
            ci_pad = ((ci + 15) // 16) * 16
            x_pad = jnp.pad(x, ((0, 0), (0, ci_pad - ci))) if ci_pad != ci else x
            xg = _gather_rows(x_pad, gi.reshape(-1)).reshape(n_i, k_i, ci_pad)
            y = _tdown(P["enc"][i]["td"], newp16, pgg, xg, n_i, k_i,
                       ci_pad, c_i, _NB_TDOWN[i])
            p16 = newp16
        idx, _ = _knn(p16, p16, k_i)
        pg3 = _gather_rows(p16, idx.reshape(-1)).reshape(n_i, k_i, 16)
        ctx = (p16, pg3, idx, n_i, k_i)
        lvl_ctx.append(ctx)
        for b in P["enc"][i]["blocks"]:
            y = _pt_block(b, ctx, y, _NB_BLOCK[i])
        ps16.append(p16)
        xs.append(y)
        x = y

    x = _tup_head(P["dec"][4]["tu"], xs[4])
    for b in P["dec"][4]["blocks"]:
        x = _pt_block(b, lvl_ctx[4], x, _NB_BLOCK[4])
    xs[4] = x

    for i in (3, 2, 1, 0):
        c_i = _PLANES[i]
        tu = P["dec"][i]["tu"]
        y1 = _dbr(xs[i], tu["l1"]["w"], tu["l1"]["b"])
        y2 = _dbr(xs[i + 1], tu["l2"]["w"], tu["l2"]["b"])
        ii, dd = _knn(ps16[i], ps16[i + 1], 3)
        y2f = _gather_rows(y2, ii.reshape(-1)).reshape(ns[i], 3 * c_i)
        x = _tup_interp(y1, y2f, dd, c_i)
        for b in P["dec"][i]["blocks"]:
            x = _pt_block(b, lvl_ctx[i], x, _NB_BLOCK[i])
        xs[i] = x

    return _cls_head(P["cls"], xs[0])
